# Initial kernel scaffold; baseline (speedup 1.0000x reference)
import jax
import jax.numpy as jnp
from jax.experimental import pallas as pl


def kernel(node_feats, edge_feats, edge_index, node_graph_ids, cat_feats, params):
    raise NotImplementedError("write your pallas kernel here")


# speedup vs baseline: 1.7807x; 1.7807x over previous
import functools

import jax
import jax.numpy as jnp
from jax import lax
from jax.experimental import pallas as pl
from jax.experimental.pallas import tpu as pltpu
from jax.experimental.pallas import tpu_sc as plsc

N = 10000
E = 320000
DN = 128
DE = 16
EMB = 300
EMBP = 320
HALF = EMBP // 2
NL = 5
B = 64
CAT = 16
HID = 256

TILES = 16
EPT = E // TILES
K = 80
NCH = EPT // K
SLAB = 624
TAIL = N - TILES * SLAB

_f32 = jnp.float32


def _split_store(o_ref, y):
    o_ref[0] = y[:, :HALF]
    o_ref[1] = y[:, HALF:]


def _matmul_body(x_ref, w_ref, b_ref, o_ref):
    y = jnp.dot(x_ref[...], w_ref[...], preferred_element_type=_f32, precision=lax.Precision.HIGHEST)
    _split_store(o_ref, y + b_ref[...])


def _matmul_split(x, w, b, rows_per_block):
    r, d = x.shape
    return pl.pallas_call(
        _matmul_body,
        grid=(r // rows_per_block,),
        in_specs=[
            pl.BlockSpec((rows_per_block, d), lambda i: (i, 0)),
            pl.BlockSpec((d, EMBP), lambda i: (0, 0)),
            pl.BlockSpec((1, EMBP), lambda i: (0, 0)),
        ],
        out_specs=pl.BlockSpec((2, rows_per_block, HALF), lambda i: (0, i, 0)),
        out_shape=jax.ShapeDtypeStruct((2, r, HALF), _f32),
    )(x, w, b)


def _mlp_body(scale_ref, h_ref, a_ref, w1_ref, b1_ref, w2_ref, b2_ref, o_ref,
              *, relu_out):
    h = jnp.concatenate([h_ref[0], h_ref[1]], axis=1)
    a = jnp.concatenate([a_ref[0], a_ref[1]], axis=1)
    z = scale_ref[0, 0] * h + a
    t = jnp.dot(z, w1_ref[...], preferred_element_type=_f32, precision=lax.Precision.HIGHEST) + b1_ref[...]
    t = jnp.maximum(t, 0.0)
    y = jnp.dot(t, w2_ref[...], preferred_element_type=_f32, precision=lax.Precision.HIGHEST) + b2_ref[...]
    if relu_out:
        y = jnp.maximum(y, 0.0)
    _split_store(o_ref, y)


def _node_mlp(scale, h, agg, w1, b1, w2, b2, relu_out):
    rows = 1000
    hdim = w1.shape[1]
    return pl.pallas_call(
        functools.partial(_mlp_body, relu_out=relu_out),
        grid=(N // rows,),
        in_specs=[
            pl.BlockSpec(memory_space=pltpu.SMEM),
            pl.BlockSpec((2, rows, HALF), lambda i: (0, i, 0)),
            pl.BlockSpec((2, rows, HALF), lambda i: (0, i, 0)),
            pl.BlockSpec((EMBP, hdim), lambda i: (0, 0)),
            pl.BlockSpec((1, hdim), lambda i: (0, 0)),
            pl.BlockSpec((hdim, EMBP), lambda i: (0, 0)),
            pl.BlockSpec((1, EMBP), lambda i: (0, 0)),
        ],
        out_specs=pl.BlockSpec((2, rows, HALF), lambda i: (0, i, 0)),
        out_shape=jax.ShapeDtypeStruct((2, N, HALF), _f32),
    )(scale, h, agg, w1, b1, w2, b2)


def _readout_body(h_ref, g_ref, ones_ref, cat_ref, wa_ref, wb_ref, b1_ref,
                  w2_ref, b2_ref, o_ref):
    h = jnp.concatenate([h_ref[0], h_ref[1]], axis=1)
    onehot = (g_ref[...] == lax.broadcasted_iota(jnp.int32, (1, B), 1))
    onehot = onehot.astype(_f32)
    sums = lax.dot_general(onehot, h, (((0,), (0,)), ((), ())),
                           preferred_element_type=_f32,
                           precision=lax.Precision.HIGHEST)
    cnt = lax.dot_general(onehot, ones_ref[...], (((0,), (0,)), ((), ())),
                          preferred_element_type=_f32, precision=lax.Precision.HIGHEST)
    gf = sums / jnp.maximum(cnt, 1.0)
    hid = (jnp.dot(gf, wa_ref[...], preferred_element_type=_f32, precision=lax.Precision.HIGHEST)
           + jnp.dot(cat_ref[...], wb_ref[...], preferred_element_type=_f32, precision=lax.Precision.HIGHEST)
           + b1_ref[...])
    hid = jnp.maximum(hid, 0.0)
    o_ref[...] = (jnp.dot(hid, w2_ref[...], preferred_element_type=_f32, precision=lax.Precision.HIGHEST)
                  + b2_ref[...])


def _readout(h, gids, ones, cat, wa, wb, b1, w2, b2):
    return pl.pallas_call(
        _readout_body,
        out_shape=jax.ShapeDtypeStruct((B, 1), _f32),
    )(h, gids, ones, cat, wa, wb, b1, w2, b2)


def _sc_body(h_hbm, e_hbm, src_hbm, dst_hbm, z_hbm, o_hbm,
             srcv, dstv, rows, ev, agg):
    c = lax.axis_index("c")
    s = lax.axis_index("s")

    pltpu.sync_copy(z_hbm.at[pl.ds(s * SLAB, SLAB)],
                    agg.at[pl.ds(s * SLAB, SLAB)])

    @pl.when(s == TILES - 1)
    def _zero_tail():
        pltpu.sync_copy(z_hbm.at[pl.ds(TILES * SLAB, TAIL)],
                        agg.at[pl.ds(TILES * SLAB, TAIL)])

    plsc.subcore_barrier()

    base = s * EPT
    hc = h_hbm.at[c]
    ec = e_hbm.at[c]

    @pl.loop(0, NCH)
    def _chunk(g):
        off = base + g * K
        pltpu.sync_copy(src_hbm.at[pl.ds(off, K)], srcv)
        pltpu.sync_copy(dst_hbm.at[pl.ds(off, K)], dstv.at[0])
        pltpu.sync_copy(hc.at[srcv], rows)
        pltpu.sync_copy(ec.at[pl.ds(off, K)], ev)

        @pl.loop(0, K)
        def _row(r):
            for j in range(HALF // 16):
                sl = (r, pl.ds(j * 16, 16))
                rows[sl] = jnp.maximum(rows[sl] + ev[sl], 0.0)

        pltpu.sync_copy(rows, agg.at[dstv.at[0]], add=True)

    plsc.subcore_barrier()
    pltpu.sync_copy(agg.at[pl.ds(s * SLAB, SLAB)],
                    o_hbm.at[c].at[pl.ds(s * SLAB, SLAB)])

    @pl.when(s == TILES - 1)
    def _out_tail():
        pltpu.sync_copy(agg.at[pl.ds(TILES * SLAB, TAIL)],
                        o_hbm.at[c].at[pl.ds(TILES * SLAB, TAIL)])


def _sc_message_pass(h, e, src, dst, zeros):
    mesh = plsc.VectorSubcoreMesh(core_axis_name="c", subcore_axis_name="s")
    kern = pl.kernel(
        _sc_body,
        out_type=jax.ShapeDtypeStruct((2, N, HALF), _f32),
        mesh=mesh,
        scratch_types=[
            pltpu.VMEM((K,), jnp.int32),
            pltpu.VMEM((1, K), jnp.int32),
            pltpu.VMEM((K, HALF), _f32),
            pltpu.VMEM((K, HALF), _f32),
            pltpu.VMEM_SHARED((N, HALF), _f32),
        ],
        compiler_params=pltpu.CompilerParams(use_tc_tiling_on_sc=False),
    )
    return kern(h, e, src, dst, zeros)


def _pad_to(x, shape):
    return jnp.pad(x, [(0, t - s) for s, t in zip(x.shape, shape)])


def kernel(node_feats, edge_feats, edge_index, node_graph_ids, cat_feats,
           params):
    src = edge_index[0]
    dst = edge_index[1]
    gids = node_graph_ids.reshape(N, 1)
    ones = jnp.ones((N, 1), _f32)
    zeros = jnp.zeros((N, HALF), _f32)

    w_in = _pad_to(params['W_in'], (DN, EMBP))
    b_in = _pad_to(params['b_in'].reshape(1, EMB), (1, EMBP))

    h = _matmul_split(node_feats, w_in, b_in, rows_per_block=1000)

    for i, lp in enumerate(params['layers']):
        we = _pad_to(lp['We'], (DE, EMBP))
        be = _pad_to(lp['be'].reshape(1, EMB), (1, EMBP))
        w1 = _pad_to(lp['W1'], (EMBP, 2 * EMB))
        b1 = lp['b1'].reshape(1, 2 * EMB)
        w2 = _pad_to(lp['W2'], (2 * EMB, EMBP))
        b2 = _pad_to(lp['b2'].reshape(1, EMB), (1, EMBP))
        scale = (1.0 + lp['eps']).reshape(1, 1)

        e = _matmul_split(edge_feats, we, be, rows_per_block=4000)
        agg = _sc_message_pass(h, e, src, dst, zeros)
        h = _node_mlp(scale, h, agg, w1, b1, w2, b2, relu_out=(i < NL - 1))

    wp1 = params['Wp1']
    wa = _pad_to(wp1[:EMB], (EMBP, HID))
    wb = wp1[EMB:]
    bp1 = params['bp1'].reshape(1, HID)
    wp2 = params['Wp2']
    bp2 = params['bp2'].reshape(1, 1)
    return _readout(h, gids, ones, cat_feats, wa, wb, bp1, wp2, bp2)


# speedup vs baseline: 1.8782x; 1.0548x over previous
import functools

import jax
import jax.numpy as jnp
from jax import lax
from jax.experimental import pallas as pl
from jax.experimental.pallas import tpu as pltpu
from jax.experimental.pallas import tpu_sc as plsc

N = 10000
E = 320000
DN = 128
DE = 16
EMB = 300
EMBP = 320
HALF = EMBP // 2
NL = 5
B = 64
CAT = 16
HID = 256

TILES = 16
EPT = E // TILES
K = 80
NCH = EPT // K
SLAB = 624
TAIL = N - TILES * SLAB

_f32 = jnp.float32


def _bdot(a, b):
    return jnp.dot(a, b, preferred_element_type=_f32)


def _split_store(o_ref, y):
    o_ref[0] = y[:, :HALF]
    o_ref[1] = y[:, HALF:]


def _matmul_body(x_ref, w_ref, b_ref, o_ref):
    y = _bdot(x_ref[...], w_ref[...])
    _split_store(o_ref, y + b_ref[...])


def _matmul_split(x, w, b, rows_per_block):
    r, d = x.shape
    return pl.pallas_call(
        _matmul_body,
        grid=(r // rows_per_block,),
        in_specs=[
            pl.BlockSpec((rows_per_block, d), lambda i: (i, 0)),
            pl.BlockSpec((d, EMBP), lambda i: (0, 0)),
            pl.BlockSpec((1, EMBP), lambda i: (0, 0)),
        ],
        out_specs=pl.BlockSpec((2, rows_per_block, HALF), lambda i: (0, i, 0)),
        out_shape=jax.ShapeDtypeStruct((2, r, HALF), _f32),
    )(x, w, b)


def _mlp_body(scale_ref, h_ref, a_ref, w1_ref, b1_ref, w2_ref, b2_ref, o_ref,
              *, relu_out):
    h = jnp.concatenate([h_ref[0], h_ref[1]], axis=1)
    a = jnp.concatenate([a_ref[0], a_ref[1]], axis=1)
    z = scale_ref[0, 0] * h + a
    t = _bdot(z, w1_ref[...]) + b1_ref[...]
    t = jnp.maximum(t, 0.0)
    y = _bdot(t, w2_ref[...]) + b2_ref[...]
    if relu_out:
        y = jnp.maximum(y, 0.0)
    _split_store(o_ref, y)


def _node_mlp(scale, h, agg, w1, b1, w2, b2, relu_out):
    rows = 1000
    hdim = w1.shape[1]
    return pl.pallas_call(
        functools.partial(_mlp_body, relu_out=relu_out),
        grid=(N // rows,),
        in_specs=[
            pl.BlockSpec(memory_space=pltpu.SMEM),
            pl.BlockSpec((2, rows, HALF), lambda i: (0, i, 0)),
            pl.BlockSpec((2, rows, HALF), lambda i: (0, i, 0)),
            pl.BlockSpec((EMBP, hdim), lambda i: (0, 0)),
            pl.BlockSpec((1, hdim), lambda i: (0, 0)),
            pl.BlockSpec((hdim, EMBP), lambda i: (0, 0)),
            pl.BlockSpec((1, EMBP), lambda i: (0, 0)),
        ],
        out_specs=pl.BlockSpec((2, rows, HALF), lambda i: (0, i, 0)),
        out_shape=jax.ShapeDtypeStruct((2, N, HALF), _f32),
    )(scale, h, agg, w1, b1, w2, b2)


def _readout_body(h_ref, g_ref, ones_ref, cat_ref, wa_ref, wb_ref, b1_ref,
                  w2_ref, b2_ref, o_ref):
    h = jnp.concatenate([h_ref[0], h_ref[1]], axis=1)
    onehot = (g_ref[...] == lax.broadcasted_iota(jnp.int32, (1, B), 1))
    onehot = onehot.astype(_f32)
    sums = lax.dot_general(onehot, h, (((0,), (0,)), ((), ())),
                           preferred_element_type=_f32,
                           precision=lax.Precision.HIGHEST)
    cnt = lax.dot_general(onehot, ones_ref[...], (((0,), (0,)), ((), ())),
                          preferred_element_type=_f32,
                          precision=lax.Precision.HIGHEST)
    gf = sums / jnp.maximum(cnt, 1.0)
    hid = _bdot(gf, wa_ref[...]) + _bdot(cat_ref[...], wb_ref[...]) + b1_ref[...]
    hid = jnp.maximum(hid, 0.0)
    o_ref[...] = _bdot(hid, w2_ref[...]) + b2_ref[...]


def _readout(h, gids, ones, cat, wa, wb, b1, w2, b2):
    return pl.pallas_call(
        _readout_body,
        out_shape=jax.ShapeDtypeStruct((B, 1), _f32),
    )(h, gids, ones, cat, wa, wb, b1, w2, b2)


def _sc_body(h_hbm, e_hbm, src_hbm, dst_hbm, z_hbm, o_hbm,
             srcv, dstv, rows, ev, agg):
    c = lax.axis_index("c")
    s = lax.axis_index("s")

    pltpu.sync_copy(z_hbm.at[pl.ds(s * SLAB, SLAB)],
                    agg.at[pl.ds(s * SLAB, SLAB)])

    @pl.when(s == TILES - 1)
    def _zero_tail():
        pltpu.sync_copy(z_hbm.at[pl.ds(TILES * SLAB, TAIL)],
                        agg.at[pl.ds(TILES * SLAB, TAIL)])

    plsc.subcore_barrier()

    base = s * EPT
    hc = h_hbm.at[c]
    ec = e_hbm.at[c]

    @pl.loop(0, NCH)
    def _chunk(g):
        off = base + g * K
        pltpu.sync_copy(src_hbm.at[pl.ds(off, K)], srcv)
        pltpu.sync_copy(dst_hbm.at[pl.ds(off, K)], dstv.at[0])
        pltpu.sync_copy(hc.at[srcv], rows)
        pltpu.sync_copy(ec.at[pl.ds(off, K)], ev)

        @plsc.parallel_loop(0, K, unroll=8)
        def _row(r):
            for j in range(HALF // 16):
                sl = (r, pl.ds(j * 16, 16))
                rows[sl] = jnp.maximum(rows[sl] + ev[sl], 0.0)

        pltpu.sync_copy(rows, agg.at[dstv.at[0]], add=True)

    plsc.subcore_barrier()
    pltpu.sync_copy(agg.at[pl.ds(s * SLAB, SLAB)],
                    o_hbm.at[c].at[pl.ds(s * SLAB, SLAB)])

    @pl.when(s == TILES - 1)
    def _out_tail():
        pltpu.sync_copy(agg.at[pl.ds(TILES * SLAB, TAIL)],
                        o_hbm.at[c].at[pl.ds(TILES * SLAB, TAIL)])


def _sc_message_pass(h, e, src, dst, zeros):
    mesh = plsc.VectorSubcoreMesh(core_axis_name="c", subcore_axis_name="s")
    kern = pl.kernel(
        _sc_body,
        out_type=jax.ShapeDtypeStruct((2, N, HALF), _f32),
        mesh=mesh,
        scratch_types=[
            pltpu.VMEM((K,), jnp.int32),
            pltpu.VMEM((1, K), jnp.int32),
            pltpu.VMEM((K, HALF), _f32),
            pltpu.VMEM((K, HALF), _f32),
            pltpu.VMEM_SHARED((N, HALF), _f32),
        ],
        compiler_params=pltpu.CompilerParams(use_tc_tiling_on_sc=False),
    )
    return kern(h, e, src, dst, zeros)


def _pad_to(x, shape):
    return jnp.pad(x, [(0, t - s) for s, t in zip(x.shape, shape)])


def kernel(node_feats, edge_feats, edge_index, node_graph_ids, cat_feats,
           params):
    src = edge_index[0]
    dst = edge_index[1]
    gids = node_graph_ids.reshape(N, 1)
    ones = jnp.ones((N, 1), _f32)
    zeros = jnp.zeros((N, HALF), _f32)

    w_in = _pad_to(params['W_in'], (DN, EMBP))
    b_in = _pad_to(params['b_in'].reshape(1, EMB), (1, EMBP))

    h = _matmul_split(node_feats, w_in, b_in, rows_per_block=1000)

    for i, lp in enumerate(params['layers']):
        we = _pad_to(lp['We'], (DE, EMBP))
        be = _pad_to(lp['be'].reshape(1, EMB), (1, EMBP))
        w1 = _pad_to(lp['W1'], (EMBP, 2 * EMB))
        b1 = lp['b1'].reshape(1, 2 * EMB)
        w2 = _pad_to(lp['W2'], (2 * EMB, EMBP))
        b2 = _pad_to(lp['b2'].reshape(1, EMB), (1, EMBP))
        scale = (1.0 + lp['eps']).reshape(1, 1)

        e = _matmul_split(edge_feats, we, be, rows_per_block=4000)
        agg = _sc_message_pass(h, e, src, dst, zeros)
        h = _node_mlp(scale, h, agg, w1, b1, w2, b2, relu_out=(i < NL - 1))

    wp1 = params['Wp1']
    wa = _pad_to(wp1[:EMB], (EMBP, HID))
    wb = wp1[EMB:]
    bp1 = params['bp1'].reshape(1, HID)
    wp2 = params['Wp2']
    bp2 = params['bp2'].reshape(1, 1)
    return _readout(h, gids, ones, cat_feats, wa, wb, bp1, wp2, bp2)


# speedup vs baseline: 2.3792x; 1.2667x over previous
import functools

import jax
import jax.numpy as jnp
from jax import lax
from jax.experimental import pallas as pl
from jax.experimental.pallas import tpu as pltpu
from jax.experimental.pallas import tpu_sc as plsc

N = 10000
E = 320000
DN = 128
DE = 16
EMB = 300
EMBP = 320
HALF = EMBP // 2
NL = 5
B = 64
CAT = 16
HID = 256

TILES = 16
EPT = E // TILES
K = 40
NCH = EPT // K
SLAB = 624
TAIL = N - TILES * SLAB

_f32 = jnp.float32


def _bdot(a, b):
    return jnp.dot(a, b, preferred_element_type=_f32)


def _split_store(o_ref, y):
    o_ref[0] = y[:, :HALF]
    o_ref[1] = y[:, HALF:]


def _matmul_body(x_ref, w_ref, b_ref, o_ref):
    y = _bdot(x_ref[...], w_ref[...])
    _split_store(o_ref, y + b_ref[...])


def _matmul_split(x, w, b, rows_per_block):
    r, d = x.shape
    return pl.pallas_call(
        _matmul_body,
        grid=(r // rows_per_block,),
        in_specs=[
            pl.BlockSpec((rows_per_block, d), lambda i: (i, 0)),
            pl.BlockSpec((d, EMBP), lambda i: (0, 0)),
            pl.BlockSpec((1, EMBP), lambda i: (0, 0)),
        ],
        out_specs=pl.BlockSpec((2, rows_per_block, HALF), lambda i: (0, i, 0)),
        out_shape=jax.ShapeDtypeStruct((2, r, HALF), _f32),
    )(x, w, b)


def _mlp_body(scale_ref, h_ref, a_ref, w1_ref, b1_ref, w2_ref, b2_ref, o_ref,
              *, relu_out):
    h = jnp.concatenate([h_ref[0], h_ref[1]], axis=1)
    a = jnp.concatenate([a_ref[0], a_ref[1]], axis=1)
    z = scale_ref[0, 0] * h + a
    t = _bdot(z, w1_ref[...]) + b1_ref[...]
    t = jnp.maximum(t, 0.0)
    y = _bdot(t, w2_ref[...]) + b2_ref[...]
    if relu_out:
        y = jnp.maximum(y, 0.0)
    _split_store(o_ref, y)


def _node_mlp(scale, h, agg, w1, b1, w2, b2, relu_out):
    rows = 1000
    hdim = w1.shape[1]
    return pl.pallas_call(
        functools.partial(_mlp_body, relu_out=relu_out),
        grid=(N // rows,),
        in_specs=[
            pl.BlockSpec(memory_space=pltpu.SMEM),
            pl.BlockSpec((2, rows, HALF), lambda i: (0, i, 0)),
            pl.BlockSpec((2, rows, HALF), lambda i: (0, i, 0)),
            pl.BlockSpec((EMBP, hdim), lambda i: (0, 0)),
            pl.BlockSpec((1, hdim), lambda i: (0, 0)),
            pl.BlockSpec((hdim, EMBP), lambda i: (0, 0)),
            pl.BlockSpec((1, EMBP), lambda i: (0, 0)),
        ],
        out_specs=pl.BlockSpec((2, rows, HALF), lambda i: (0, i, 0)),
        out_shape=jax.ShapeDtypeStruct((2, N, HALF), _f32),
    )(scale, h, agg, w1, b1, w2, b2)


def _readout_body(h_ref, g_ref, ones_ref, cat_ref, wa_ref, wb_ref, b1_ref,
                  w2_ref, b2_ref, o_ref):
    h = jnp.concatenate([h_ref[0], h_ref[1]], axis=1)
    onehot = (g_ref[...] == lax.broadcasted_iota(jnp.int32, (1, B), 1))
    onehot = onehot.astype(_f32)
    sums = lax.dot_general(onehot, h, (((0,), (0,)), ((), ())),
                           preferred_element_type=_f32,
                           precision=lax.Precision.HIGHEST)
    cnt = lax.dot_general(onehot, ones_ref[...], (((0,), (0,)), ((), ())),
                          preferred_element_type=_f32,
                          precision=lax.Precision.HIGHEST)
    gf = sums / jnp.maximum(cnt, 1.0)
    hid = _bdot(gf, wa_ref[...]) + _bdot(cat_ref[...], wb_ref[...]) + b1_ref[...]
    hid = jnp.maximum(hid, 0.0)
    o_ref[...] = _bdot(hid, w2_ref[...]) + b2_ref[...]


def _readout(h, gids, ones, cat, wa, wb, b1, w2, b2):
    return pl.pallas_call(
        _readout_body,
        out_shape=jax.ShapeDtypeStruct((B, 1), _f32),
    )(h, gids, ones, cat, wa, wb, b1, w2, b2)


def _sc_body(h_hbm, e_hbm, src_hbm, dst_hbm, z_hbm, o_hbm,
             srcv, dstv, rows, ev, agg,
             sem_src, sem_dst, sem_e, sem_g, sem_s):
    c = lax.axis_index("c")
    s = lax.axis_index("s")

    pltpu.sync_copy(z_hbm.at[pl.ds(s * SLAB, SLAB)],
                    agg.at[pl.ds(s * SLAB, SLAB)])

    @pl.when(s == TILES - 1)
    def _zero_tail():
        pltpu.sync_copy(z_hbm.at[pl.ds(TILES * SLAB, TAIL)],
                        agg.at[pl.ds(TILES * SLAB, TAIL)])

    plsc.subcore_barrier()

    base = s * EPT
    hc = h_hbm.at[c]
    ec = e_hbm.at[c]

    def start(b, g):
        off = base + g * K
        pltpu.async_copy(src_hbm.at[pl.ds(off, K)], srcv.at[b], sem_src.at[b])
        pltpu.async_copy(ec.at[pl.ds(off, K)], ev.at[b], sem_e.at[b])

    def drain_scatter(b):
        pltpu.make_async_copy(rows.at[b], agg.at[dstv.at[b]],
                              sem_s.at[b]).wait()

    def gather(b):
        pltpu.make_async_copy(src_hbm.at[pl.ds(0, K)], srcv.at[b],
                              sem_src.at[b]).wait()
        pltpu.async_copy(hc.at[srcv.at[b]], rows.at[b], sem_g.at[b])

    def process(b, g):
        pltpu.make_async_copy(hc.at[srcv.at[b]], rows.at[b],
                              sem_g.at[b]).wait()
        pltpu.make_async_copy(ec.at[pl.ds(0, K)], ev.at[b], sem_e.at[b]).wait()
        off = base + g * K
        pltpu.async_copy(dst_hbm.at[pl.ds(off, K)], dstv.at[b], sem_dst.at[b])
        rb, eb = rows.at[b], ev.at[b]

        @plsc.parallel_loop(0, K, unroll=8)
        def _row(r):
            for j in range(HALF // 16):
                sl = (r, pl.ds(j * 16, 16))
                rb[sl] = jnp.maximum(rb[sl] + eb[sl], 0.0)

        pltpu.make_async_copy(dst_hbm.at[pl.ds(0, K)], dstv.at[b],
                              sem_dst.at[b]).wait()
        pltpu.async_copy(rows.at[b], agg.at[dstv.at[b]], sem_s.at[b], add=True)

    start(0, 0)
    start(1, 1)
    gather(0)

    @pl.loop(0, NCH, step=2)
    def _pair(g):
        @pl.when(g > 0)
        def _():
            drain_scatter(1)

        gather(1)
        process(0, g)
        @pl.when(g + 2 < NCH)
        def _():
            start(0, g + 2)

        process(1, g + 1)
        drain_scatter(0)
        @pl.when(g + 2 < NCH)
        def _():
            gather(0)
            start(1, g + 3)

    drain_scatter(1)

    plsc.subcore_barrier()
    pltpu.sync_copy(agg.at[pl.ds(s * SLAB, SLAB)],
                    o_hbm.at[c].at[pl.ds(s * SLAB, SLAB)])

    @pl.when(s == TILES - 1)
    def _out_tail():
        pltpu.sync_copy(agg.at[pl.ds(TILES * SLAB, TAIL)],
                        o_hbm.at[c].at[pl.ds(TILES * SLAB, TAIL)])


def _sc_message_pass(h, e, src, dst, zeros):
    mesh = plsc.VectorSubcoreMesh(core_axis_name="c", subcore_axis_name="s")
    kern = pl.kernel(
        _sc_body,
        out_type=jax.ShapeDtypeStruct((2, N, HALF), _f32),
        mesh=mesh,
        scratch_types=[
            pltpu.VMEM((2, K), jnp.int32),
            pltpu.VMEM((2, K), jnp.int32),
            pltpu.VMEM((2, K, HALF), _f32),
            pltpu.VMEM((2, K, HALF), _f32),
            pltpu.VMEM_SHARED((N, HALF), _f32),
            pltpu.SemaphoreType.DMA((2,)),
            pltpu.SemaphoreType.DMA((2,)),
            pltpu.SemaphoreType.DMA((2,)),
            pltpu.SemaphoreType.DMA((2,)),
            pltpu.SemaphoreType.DMA((2,)),
        ],
        compiler_params=pltpu.CompilerParams(use_tc_tiling_on_sc=False),
    )
    return kern(h, e, src, dst, zeros)


def _pad_to(x, shape):
    return jnp.pad(x, [(0, t - s) for s, t in zip(x.shape, shape)])


def kernel(node_feats, edge_feats, edge_index, node_graph_ids, cat_feats,
           params):
    src = edge_index[0]
    dst = edge_index[1]
    gids = node_graph_ids.reshape(N, 1)
    ones = jnp.ones((N, 1), _f32)
    zeros = jnp.zeros((N, HALF), _f32)

    w_in = _pad_to(params['W_in'], (DN, EMBP))
    b_in = _pad_to(params['b_in'].reshape(1, EMB), (1, EMBP))

    h = _matmul_split(node_feats, w_in, b_in, rows_per_block=1000)

    for i, lp in enumerate(params['layers']):
        we = _pad_to(lp['We'], (DE, EMBP))
        be = _pad_to(lp['be'].reshape(1, EMB), (1, EMBP))
        w1 = _pad_to(lp['W1'], (EMBP, 2 * EMB))
        b1 = lp['b1'].reshape(1, 2 * EMB)
        w2 = _pad_to(lp['W2'], (2 * EMB, EMBP))
        b2 = _pad_to(lp['b2'].reshape(1, EMB), (1, EMBP))
        scale = (1.0 + lp['eps']).reshape(1, 1)

        e = _matmul_split(edge_feats, we, be, rows_per_block=4000)
        agg = _sc_message_pass(h, e, src, dst, zeros)
        h = _node_mlp(scale, h, agg, w1, b1, w2, b2, relu_out=(i < NL - 1))

    wp1 = params['Wp1']
    wa = _pad_to(wp1[:EMB], (EMBP, HID))
    wb = wp1[EMB:]
    bp1 = params['bp1'].reshape(1, HID)
    wp2 = params['Wp2']
    bp2 = params['bp2'].reshape(1, 1)
    return _readout(h, gids, ones, cat_feats, wa, wb, bp1, wp2, bp2)


# speedup vs baseline: 2.6972x; 1.1337x over previous
import functools

import jax
import jax.numpy as jnp
from jax import lax
from jax.experimental import pallas as pl
from jax.experimental.pallas import tpu as pltpu
from jax.experimental.pallas import tpu_sc as plsc

N = 10000
E = 320000
DN = 128
DE = 16
EMB = 300
EMBP = 320
HALF = EMBP // 2
NL = 5
B = 64
CAT = 16
HID = 256

TILES = 16
EPT = E // TILES
K = 40
NCH = EPT // K
SLAB = 624
TAIL = N - TILES * SLAB

_f32 = jnp.float32


def _bdot(a, b):
    return jnp.dot(a, b, preferred_element_type=_f32)


def _split_store(o_ref, y):
    o_ref[0] = y[:, :HALF]
    o_ref[1] = y[:, HALF:]


def _matmul_body(x_ref, w_ref, b_ref, o_ref):
    y = _bdot(x_ref[...], w_ref[...])
    _split_store(o_ref, y + b_ref[...])


def _edges_body(x_ref, w_ref, b_ref, *o_refs):
    y = _bdot(x_ref[...], w_ref[...]) + b_ref[...]
    for i, o_ref in enumerate(o_refs):
        _split_store(o_ref, y[:, i * EMBP:(i + 1) * EMBP])


def _edge_proj_all(x, w, b):
    rows = 2000
    nl = w.shape[1] // EMBP
    return pl.pallas_call(
        _edges_body,
        grid=(E // rows,),
        in_specs=[
            pl.BlockSpec((rows, DE), lambda i: (i, 0)),
            pl.BlockSpec((DE, nl * EMBP), lambda i: (0, 0)),
            pl.BlockSpec((1, nl * EMBP), lambda i: (0, 0)),
        ],
        out_specs=[pl.BlockSpec((2, rows, HALF), lambda i: (0, i, 0))] * nl,
        out_shape=[jax.ShapeDtypeStruct((2, E, HALF), _f32)] * nl,
    )(x, w, b)


def _matmul_split(x, w, b, rows_per_block):
    r, d = x.shape
    return pl.pallas_call(
        _matmul_body,
        grid=(r // rows_per_block,),
        in_specs=[
            pl.BlockSpec((rows_per_block, d), lambda i: (i, 0)),
            pl.BlockSpec((d, EMBP), lambda i: (0, 0)),
            pl.BlockSpec((1, EMBP), lambda i: (0, 0)),
        ],
        out_specs=pl.BlockSpec((2, rows_per_block, HALF), lambda i: (0, i, 0)),
        out_shape=jax.ShapeDtypeStruct((2, r, HALF), _f32),
    )(x, w, b)


def _mlp_body(scale_ref, h_ref, a_ref, w1_ref, b1_ref, w2_ref, b2_ref, o_ref,
              *, relu_out):
    h = jnp.concatenate([h_ref[0], h_ref[1]], axis=1)
    a = jnp.concatenate([a_ref[0], a_ref[1]], axis=1)
    z = scale_ref[0, 0] * h + a
    t = _bdot(z, w1_ref[...]) + b1_ref[...]
    t = jnp.maximum(t, 0.0)
    y = _bdot(t, w2_ref[...]) + b2_ref[...]
    if relu_out:
        y = jnp.maximum(y, 0.0)
    _split_store(o_ref, y)


def _node_mlp(scale, h, agg, w1, b1, w2, b2, relu_out):
    rows = 2000
    hdim = w1.shape[1]
    return pl.pallas_call(
        functools.partial(_mlp_body, relu_out=relu_out),
        grid=(N // rows,),
        in_specs=[
            pl.BlockSpec(memory_space=pltpu.SMEM),
            pl.BlockSpec((2, rows, HALF), lambda i: (0, i, 0)),
            pl.BlockSpec((2, rows, HALF), lambda i: (0, i, 0)),
            pl.BlockSpec((EMBP, hdim), lambda i: (0, 0)),
            pl.BlockSpec((1, hdim), lambda i: (0, 0)),
            pl.BlockSpec((hdim, EMBP), lambda i: (0, 0)),
            pl.BlockSpec((1, EMBP), lambda i: (0, 0)),
        ],
        out_specs=pl.BlockSpec((2, rows, HALF), lambda i: (0, i, 0)),
        out_shape=jax.ShapeDtypeStruct((2, N, HALF), _f32),
    )(scale, h, agg, w1, b1, w2, b2)


def _readout_body(h_ref, g_ref, ones_ref, cat_ref, wa_ref, wb_ref, b1_ref,
                  w2_ref, b2_ref, o_ref):
    h = jnp.concatenate([h_ref[0], h_ref[1]], axis=1)
    onehot = (g_ref[...] == lax.broadcasted_iota(jnp.int32, (1, B), 1))
    onehot = onehot.astype(_f32)
    sums = lax.dot_general(onehot, h, (((0,), (0,)), ((), ())),
                           preferred_element_type=_f32,
                           precision=lax.Precision.HIGHEST)
    cnt = lax.dot_general(onehot, ones_ref[...], (((0,), (0,)), ((), ())),
                          preferred_element_type=_f32,
                          precision=lax.Precision.HIGHEST)
    gf = sums / jnp.maximum(cnt, 1.0)
    hid = _bdot(gf, wa_ref[...]) + _bdot(cat_ref[...], wb_ref[...]) + b1_ref[...]
    hid = jnp.maximum(hid, 0.0)
    o_ref[...] = _bdot(hid, w2_ref[...]) + b2_ref[...]


def _readout(h, gids, ones, cat, wa, wb, b1, w2, b2):
    return pl.pallas_call(
        _readout_body,
        out_shape=jax.ShapeDtypeStruct((B, 1), _f32),
    )(h, gids, ones, cat, wa, wb, b1, w2, b2)


def _sc_body(h_hbm, e_hbm, src_hbm, dst_hbm, z_hbm, o_hbm,
             srcv, dstv, rows, ev, agg,
             sem_src, sem_dst, sem_e, sem_g, sem_s):
    c = lax.axis_index("c")
    s = lax.axis_index("s")

    pltpu.sync_copy(z_hbm.at[pl.ds(s * SLAB, SLAB)],
                    agg.at[pl.ds(s * SLAB, SLAB)])

    @pl.when(s == TILES - 1)
    def _zero_tail():
        pltpu.sync_copy(z_hbm.at[pl.ds(TILES * SLAB, TAIL)],
                        agg.at[pl.ds(TILES * SLAB, TAIL)])

    plsc.subcore_barrier()

    base = s * EPT
    hc = h_hbm.at[c]
    ec = e_hbm.at[c]

    def start(b, g):
        off = base + g * K
        pltpu.async_copy(src_hbm.at[pl.ds(off, K)], srcv.at[b], sem_src.at[b])
        pltpu.async_copy(ec.at[pl.ds(off, K)], ev.at[b], sem_e.at[b])

    def drain_scatter(b):
        pltpu.make_async_copy(rows.at[b], agg.at[dstv.at[b]],
                              sem_s.at[b]).wait()

    def gather(b):
        pltpu.make_async_copy(src_hbm.at[pl.ds(0, K)], srcv.at[b],
                              sem_src.at[b]).wait()
        pltpu.async_copy(hc.at[srcv.at[b]], rows.at[b], sem_g.at[b])

    def process(b, g):
        pltpu.make_async_copy(hc.at[srcv.at[b]], rows.at[b],
                              sem_g.at[b]).wait()
        pltpu.make_async_copy(ec.at[pl.ds(0, K)], ev.at[b], sem_e.at[b]).wait()
        off = base + g * K
        pltpu.async_copy(dst_hbm.at[pl.ds(off, K)], dstv.at[b], sem_dst.at[b])
        rb, eb = rows.at[b], ev.at[b]

        @plsc.parallel_loop(0, K, unroll=8)
        def _row(r):
            for j in range(HALF // 16):
                sl = (r, pl.ds(j * 16, 16))
                rb[sl] = jnp.maximum(rb[sl] + eb[sl], 0.0)

        pltpu.make_async_copy(dst_hbm.at[pl.ds(0, K)], dstv.at[b],
                              sem_dst.at[b]).wait()
        pltpu.async_copy(rows.at[b], agg.at[dstv.at[b]], sem_s.at[b], add=True)

    start(0, 0)
    start(1, 1)
    gather(0)

    @pl.loop(0, NCH, step=2)
    def _pair(g):
        @pl.when(g > 0)
        def _():
            drain_scatter(1)

        gather(1)
        process(0, g)
        @pl.when(g + 2 < NCH)
        def _():
            start(0, g + 2)

        process(1, g + 1)
        drain_scatter(0)
        @pl.when(g + 2 < NCH)
        def _():
            gather(0)
            start(1, g + 3)

    drain_scatter(1)

    plsc.subcore_barrier()
    pltpu.sync_copy(agg.at[pl.ds(s * SLAB, SLAB)],
                    o_hbm.at[c].at[pl.ds(s * SLAB, SLAB)])

    @pl.when(s == TILES - 1)
    def _out_tail():
        pltpu.sync_copy(agg.at[pl.ds(TILES * SLAB, TAIL)],
                        o_hbm.at[c].at[pl.ds(TILES * SLAB, TAIL)])


def _sc_message_pass(h, e, src, dst, zeros):
    mesh = plsc.VectorSubcoreMesh(core_axis_name="c", subcore_axis_name="s")
    kern = pl.kernel(
        _sc_body,
        out_type=jax.ShapeDtypeStruct((2, N, HALF), _f32),
        mesh=mesh,
        scratch_types=[
            pltpu.VMEM((2, K), jnp.int32),
            pltpu.VMEM((2, K), jnp.int32),
            pltpu.VMEM((2, K, HALF), _f32),
            pltpu.VMEM((2, K, HALF), _f32),
            pltpu.VMEM_SHARED((N, HALF), _f32),
            pltpu.SemaphoreType.DMA((2,)),
            pltpu.SemaphoreType.DMA((2,)),
            pltpu.SemaphoreType.DMA((2,)),
            pltpu.SemaphoreType.DMA((2,)),
            pltpu.SemaphoreType.DMA((2,)),
        ],
        compiler_params=pltpu.CompilerParams(use_tc_tiling_on_sc=False),
    )
    return kern(h, e, src, dst, zeros)


def _pad_to(x, shape):
    return jnp.pad(x, [(0, t - s) for s, t in zip(x.shape, shape)])


def kernel(node_feats, edge_feats, edge_index, node_graph_ids, cat_feats,
           params):
    src = edge_index[0]
    dst = edge_index[1]
    gids = node_graph_ids.reshape(N, 1)
    ones = jnp.ones((N, 1), _f32)
    zeros = jnp.zeros((N, HALF), _f32)

    w_in = _pad_to(params['W_in'], (DN, EMBP))
    b_in = _pad_to(params['b_in'].reshape(1, EMB), (1, EMBP))

    h = _matmul_split(node_feats, w_in, b_in, rows_per_block=1000)

    we_all = jnp.concatenate(
        [_pad_to(lp['We'], (DE, EMBP)) for lp in params['layers']], axis=1)
    be_all = jnp.concatenate(
        [_pad_to(lp['be'].reshape(1, EMB), (1, EMBP))
         for lp in params['layers']], axis=1)
    es = _edge_proj_all(edge_feats, we_all, be_all)

    for i, lp in enumerate(params['layers']):
        w1 = _pad_to(lp['W1'], (EMBP, 2 * EMB))
        b1 = lp['b1'].reshape(1, 2 * EMB)
        w2 = _pad_to(lp['W2'], (2 * EMB, EMBP))
        b2 = _pad_to(lp['b2'].reshape(1, EMB), (1, EMBP))
        scale = (1.0 + lp['eps']).reshape(1, 1)

        agg = _sc_message_pass(h, es[i], src, dst, zeros)
        h = _node_mlp(scale, h, agg, w1, b1, w2, b2, relu_out=(i < NL - 1))

    wp1 = params['Wp1']
    wa = _pad_to(wp1[:EMB], (EMBP, HID))
    wb = wp1[EMB:]
    bp1 = params['bp1'].reshape(1, HID)
    wp2 = params['Wp2']
    bp2 = params['bp2'].reshape(1, 1)
    return _readout(h, gids, ones, cat_feats, wa, wb, bp1, wp2, bp2)
